# baseline (speedup 1.0000x reference)
import functools

import jax
import jax.numpy as jnp
from jax import lax
from jax.experimental import pallas as pl
from jax.experimental.pallas import tpu as pltpu
from jax.experimental.pallas import tpu_sc as plsc

_N = 4096
_D = 1024
_E = 8
_F = 2048
_TM = 256
_NB = _N // _TM
_NT = 23
_NPAD = _NT * _TM
_FB = 512
_NF = _F // _FB
_NW = 32
_RPW = _N // _NW
_CH = 64
_NCH = _RPW // _CH


def _router_body(x_ref, rw_ref, mask_ref, g_ref, e_ref, rank_ref, cnt_ref, acc):
    i = pl.program_id(0)

    @pl.when(i == 0)
    def _():
        acc[...] = jnp.zeros_like(acc)

    xb = x_ref[...]
    logits = lax.dot_general(xb, rw_ref[...], (((1,), (1,)), ((), ())),
                             preferred_element_type=jnp.float32)
    m = jnp.max(logits, axis=-1, keepdims=True)
    ex = jnp.exp(logits - m)
    s = jnp.sum(ex, axis=-1, keepdims=True)
    probs = ex / s
    pm = jnp.max(probs, axis=-1, keepdims=True)
    lane = lax.broadcasted_iota(jnp.int32, (_TM, _E), 1).astype(jnp.float32)
    cand = jnp.where(probs >= pm, lane, float(_E))
    e = jnp.min(cand, axis=-1, keepdims=True)
    oh = (lane == e).astype(jnp.float32)
    r0 = lax.broadcasted_iota(jnp.int32, (_TM, _TM), 0)
    c0 = lax.broadcasted_iota(jnp.int32, (_TM, _TM), 1)
    tri = (c0 < r0).astype(jnp.float32)
    rank_all = jnp.dot(tri, oh, preferred_element_type=jnp.float32)
    base = acc[...]
    rank_tok = jnp.sum((rank_all + base) * oh, axis=-1, keepdims=True)
    gate = pm * mask_ref[...]
    g_ref[...] = jnp.broadcast_to(gate, (_TM, 128))
    e_ref[...] = e
    rank_ref[...] = rank_tok
    acc[...] = base + jnp.sum(oh, axis=0, keepdims=True)

    @pl.when(i == _NB - 1)
    def _():
        cnt_ref[...] = acc[...]


def _plan_body(cnt_ref, e_ref, rank_ref, p_ref, te_ref, act_ref, offs_s):
    i = pl.program_id(0)

    @pl.when(i == 0)
    def _():
        c = cnt_ref[...]
        a = jnp.ceil(c * (1.0 / _TM)) * float(_TM)
        r0 = lax.broadcasted_iota(jnp.int32, (_E, _E), 0)
        c0 = lax.broadcasted_iota(jnp.int32, (_E, _E), 1)
        triu = (r0 < c0).astype(jnp.float32)
        offs = jnp.dot(a, triu, preferred_element_type=jnp.float32)
        offs_s[...] = offs
        ends = offs + a
        eye = (r0 == c0).astype(jnp.float32)
        ends_col = lax.dot_general(eye, ends, (((1,), (1,)), ((), ())),
                                   preferred_element_type=jnp.float32)
        tt = lax.broadcasted_iota(jnp.int32, (_E, 128), 1).astype(jnp.float32) * float(_TM)
        te = jnp.sum((ends_col <= tt).astype(jnp.float32), axis=0, keepdims=True)
        te_ref[...] = jnp.minimum(te, float(_E - 1)).astype(jnp.int32)
        total = jnp.sum(a, axis=-1, keepdims=True)
        trow = lax.broadcasted_iota(jnp.int32, (1, 128), 1).astype(jnp.float32) * float(_TM)
        act_ref[...] = (trow < total).astype(jnp.int32)

    e = e_ref[...]
    lane = lax.broadcasted_iota(jnp.int32, (_TM, _E), 1).astype(jnp.float32)
    oh = (lane == e).astype(jnp.float32)
    off_tok = jnp.sum(oh * offs_s[...], axis=-1, keepdims=True)
    p_ref[...] = (off_tok + rank_ref[...]).astype(jnp.int32)


def _ffn_body(te_ref, act_ref, xs_ref, gs_ref, w1_ref, w2_ref, o_ref, w1c, w2c, eid_s):
    t = pl.program_id(0)

    @pl.when(t == 0)
    def _():
        eid_s[0] = -1

    e = te_ref[t]
    active = act_ref[t] == 1

    @pl.when(jnp.logical_and(active, eid_s[0] != e))
    def _():
        w1c[...] = w1_ref[0].astype(jnp.bfloat16)
        w2c[...] = w2_ref[0].astype(jnp.bfloat16)
        eid_s[0] = e

    @pl.when(active)
    def _():
        xb16 = (xs_ref[...] * gs_ref[:, 0:1]).astype(jnp.bfloat16)
        h = jnp.dot(xb16, w1c[...], preferred_element_type=jnp.float32)
        h16 = jnp.maximum(h, 0.0).astype(jnp.bfloat16)
        o_ref[...] = jnp.dot(h16, w2c[...], preferred_element_type=jnp.float32)


@functools.cache
def _sc_kernels():
    mesh = plsc.VectorSubcoreMesh(core_axis_name="c", subcore_axis_name="s")
    scratch = [
        pltpu.VMEM((_NCH, _CH), jnp.int32),
        pltpu.VMEM((_CH, _D), jnp.float32),
        pltpu.SemaphoreType.DMA,
    ]

    @functools.partial(
        pl.kernel,
        out_type=(
            jax.ShapeDtypeStruct((_NPAD, _D), jnp.float32),
            jax.ShapeDtypeStruct((_NPAD, 128), jnp.float32),
        ),
        mesh=mesh,
        scratch_types=scratch + [
            pltpu.VMEM((_CH, 128), jnp.float32),
            pltpu.SemaphoreType.DMA,
        ],
    )
    def sc_scatter(x_hbm, g_hbm, p_hbm, xs_hbm, gs_hbm, idx_v, rows_v, sem,
                   g_v, sem2):
        wid = lax.axis_index("s") * 2 + lax.axis_index("c")
        base = wid * _RPW
        pltpu.sync_copy(p_hbm.at[wid], idx_v)
        for c in range(_NCH):
            pltpu.sync_copy(x_hbm.at[pl.ds(base + c * _CH, _CH)], rows_v)
            pltpu.sync_copy(g_hbm.at[pl.ds(base + c * _CH, _CH)], g_v)
            cp1 = pltpu.async_copy(rows_v, xs_hbm.at[idx_v.at[c]], sem)
            cp2 = pltpu.async_copy(g_v, gs_hbm.at[idx_v.at[c]], sem2)
            cp1.wait()
            cp2.wait()

    @functools.partial(
        pl.kernel,
        out_type=jax.ShapeDtypeStruct((_N, _D), jnp.float32),
        mesh=mesh,
        scratch_types=scratch,
    )
    def sc_gather(os_hbm, p_hbm, y_hbm, idx_v, rows_v, sem):
        wid = lax.axis_index("s") * 2 + lax.axis_index("c")
        base = wid * _RPW
        pltpu.sync_copy(p_hbm.at[wid], idx_v)
        for c in range(_NCH):
            pltpu.async_copy(os_hbm.at[idx_v.at[c]], rows_v, sem).wait()
            pltpu.sync_copy(rows_v, y_hbm.at[pl.ds(base + c * _CH, _CH)])

    return sc_scatter, sc_gather


def kernel(x, token_mask, router_w, w1, w2):
    Bc, Tc, Dc = x.shape
    xf = x.reshape(_N, _D)
    maskf = token_mask.reshape(_N, 1).astype(jnp.float32)

    router_call = pl.pallas_call(
        _router_body,
        grid=(_NB,),
        in_specs=[
            pl.BlockSpec((_TM, _D), lambda i: (i, 0)),
            pl.BlockSpec((_E, _D), lambda i: (0, 0)),
            pl.BlockSpec((_TM, 1), lambda i: (i, 0)),
        ],
        out_specs=[
            pl.BlockSpec((_TM, 128), lambda i: (i, 0)),
            pl.BlockSpec((_TM, 1), lambda i: (i, 0)),
            pl.BlockSpec((_TM, 1), lambda i: (i, 0)),
            pl.BlockSpec((1, _E), lambda i: (0, 0)),
        ],
        out_shape=[
            jax.ShapeDtypeStruct((_N, 128), jnp.float32),
            jax.ShapeDtypeStruct((_N, 1), jnp.float32),
            jax.ShapeDtypeStruct((_N, 1), jnp.float32),
            jax.ShapeDtypeStruct((1, _E), jnp.float32),
        ],
        scratch_shapes=[pltpu.VMEM((1, _E), jnp.float32)],
    )
    g16, ef, rank, counts = router_call(xf, router_w, maskf)

    plan_call = pl.pallas_call(
        _plan_body,
        grid=(_NB,),
        in_specs=[
            pl.BlockSpec((1, _E), lambda i: (0, 0)),
            pl.BlockSpec((_TM, 1), lambda i: (i, 0)),
            pl.BlockSpec((_TM, 1), lambda i: (i, 0)),
        ],
        out_specs=[
            pl.BlockSpec((_TM, 1), lambda i: (i, 0)),
            pl.BlockSpec((1, 128), lambda i: (0, 0)),
            pl.BlockSpec((1, 128), lambda i: (0, 0)),
        ],
        out_shape=[
            jax.ShapeDtypeStruct((_N, 1), jnp.int32),
            jax.ShapeDtypeStruct((1, 128), jnp.int32),
            jax.ShapeDtypeStruct((1, 128), jnp.int32),
        ],
        scratch_shapes=[pltpu.VMEM((1, _E), jnp.float32)],
    )
    p, te_pad, act_pad = plan_call(counts, ef, rank)

    p3 = p.reshape(_NW, _NCH, _CH)
    te = te_pad.reshape(128)[:_NT]
    act = act_pad.reshape(128)[:_NT]

    sc_scatter, sc_gather = _sc_kernels()
    xs, gs = sc_scatter(xf, g16, p3)

    ffn_call = pl.pallas_call(
        _ffn_body,
        grid_spec=pltpu.PrefetchScalarGridSpec(
            num_scalar_prefetch=2,
            grid=(_NT,),
            in_specs=[
                pl.BlockSpec((_TM, _D), lambda t, te_r, act_r: (t, 0)),
                pl.BlockSpec((_TM, 128), lambda t, te_r, act_r: (t, 0)),
                pl.BlockSpec((1, _D, _F), lambda t, te_r, act_r: (te_r[t], 0, 0)),
                pl.BlockSpec((1, _F, _D), lambda t, te_r, act_r: (te_r[t], 0, 0)),
            ],
            out_specs=pl.BlockSpec((_TM, _D), lambda t, te_r, act_r: (t, 0)),
            scratch_shapes=[
                pltpu.VMEM((_D, _F), jnp.bfloat16),
                pltpu.VMEM((_F, _D), jnp.bfloat16),
                pltpu.SMEM((1,), jnp.int32),
            ],
        ),
        out_shape=jax.ShapeDtypeStruct((_NPAD, _D), jnp.float32),
        compiler_params=pltpu.CompilerParams(
            dimension_semantics=("arbitrary",)),
    )
    osrt = ffn_call(te, act, xs, gs, w1, w2)

    y = sc_gather(osrt, p3)
    return y.reshape(Bc, Tc, Dc)


# speedup vs baseline: 1.0332x; 1.0332x over previous
import functools

import jax
import jax.numpy as jnp
from jax import lax
from jax.experimental import pallas as pl
from jax.experimental.pallas import tpu as pltpu
from jax.experimental.pallas import tpu_sc as plsc

_N = 4096
_D = 1024
_E = 8
_F = 2048
_TM = 256
_NB = _N // _TM
_NT = 23
_NPAD = _NT * _TM
_FB = 512
_NF = _F // _FB
_NW = 32
_RPW = _N // _NW
_CH = 64
_NCH = _RPW // _CH


def _route_plan_body(x_ref, rw_ref, mask_ref, g_ref, p_ref, te_ref, act_ref,
                     e_all, r_all, acc, offs_s):
    i = pl.program_id(0)

    @pl.when(i == 0)
    def _():
        acc[...] = jnp.zeros_like(acc)

    lane = lax.broadcasted_iota(jnp.int32, (_TM, _E), 1).astype(jnp.float32)

    @pl.when(i < _NB)
    def _():
        xb = x_ref[...]
        logits = lax.dot_general(xb, rw_ref[...], (((1,), (1,)), ((), ())),
                                 preferred_element_type=jnp.float32)
        m = jnp.max(logits, axis=-1, keepdims=True)
        ex = jnp.exp(logits - m)
        s = jnp.sum(ex, axis=-1, keepdims=True)
        probs = ex / s
        pm = jnp.max(probs, axis=-1, keepdims=True)
        cand = jnp.where(probs >= pm, lane, float(_E))
        e = jnp.min(cand, axis=-1, keepdims=True)
        oh = (lane == e).astype(jnp.float32)
        r0 = lax.broadcasted_iota(jnp.int32, (_TM, _TM), 0)
        c0 = lax.broadcasted_iota(jnp.int32, (_TM, _TM), 1)
        tri = (c0 < r0).astype(jnp.float32)
        rank_all = jnp.dot(tri, oh, preferred_element_type=jnp.float32)
        base = acc[...]
        rank_tok = jnp.sum((rank_all + base) * oh, axis=-1, keepdims=True)
        gate = pm * mask_ref[...]
        g_ref[...] = jnp.broadcast_to(gate, (_TM, 128))
        e_all[pl.ds(i * _TM, _TM), :] = e
        r_all[pl.ds(i * _TM, _TM), :] = rank_tok
        acc[...] = base + jnp.sum(oh, axis=0, keepdims=True)

    @pl.when(i == _NB)
    def _():
        c = acc[...]
        a = jnp.ceil(c * (1.0 / _TM)) * float(_TM)
        r0 = lax.broadcasted_iota(jnp.int32, (_E, _E), 0)
        c0 = lax.broadcasted_iota(jnp.int32, (_E, _E), 1)
        triu = (r0 < c0).astype(jnp.float32)
        offs = jnp.dot(a, triu, preferred_element_type=jnp.float32)
        offs_s[...] = offs
        ends = offs + a
        eye = (r0 == c0).astype(jnp.float32)
        ends_col = lax.dot_general(eye, ends, (((1,), (1,)), ((), ())),
                                   preferred_element_type=jnp.float32)
        tt = lax.broadcasted_iota(jnp.int32, (_E, 128), 1).astype(jnp.float32) * float(_TM)
        te = jnp.sum((ends_col <= tt).astype(jnp.float32), axis=0, keepdims=True)
        te_ref[...] = jnp.minimum(te, float(_E - 1)).astype(jnp.int32)
        total = jnp.sum(a, axis=-1, keepdims=True)
        trow = lax.broadcasted_iota(jnp.int32, (1, 128), 1).astype(jnp.float32) * float(_TM)
        act_ref[...] = (trow < total).astype(jnp.int32)

    @pl.when(i >= _NB)
    def _():
        j = i - _NB
        e = e_all[pl.ds(j * _TM, _TM), :]
        oh = (lane == e).astype(jnp.float32)
        off_tok = jnp.sum(oh * offs_s[...], axis=-1, keepdims=True)
        p_ref[...] = (off_tok + r_all[pl.ds(j * _TM, _TM), :]).astype(jnp.int32)


def _ffn_body(te_ref, act_ref, xs_ref, gs_ref, w1_ref, w2_ref, o_ref, w1c, w2c, eid_s):
    t = pl.program_id(0)

    @pl.when(t == 0)
    def _():
        eid_s[0] = -1

    e = te_ref[t]
    active = act_ref[t] == 1

    @pl.when(jnp.logical_and(active, eid_s[0] != e))
    def _():
        w1c[...] = w1_ref[0].astype(jnp.bfloat16)
        w2c[...] = w2_ref[0].astype(jnp.bfloat16)
        eid_s[0] = e

    @pl.when(active)
    def _():
        xb16 = (xs_ref[...] * gs_ref[:, 0:1]).astype(jnp.bfloat16)
        h = jnp.dot(xb16, w1c[...], preferred_element_type=jnp.float32)
        h16 = jnp.maximum(h, 0.0).astype(jnp.bfloat16)
        o_ref[...] = jnp.dot(h16, w2c[...], preferred_element_type=jnp.float32)


@functools.cache
def _sc_kernels():
    mesh = plsc.VectorSubcoreMesh(core_axis_name="c", subcore_axis_name="s")
    scratch = [
        pltpu.VMEM((_NCH, _CH), jnp.int32),
        pltpu.VMEM((_CH, _D), jnp.float32),
        pltpu.SemaphoreType.DMA,
    ]

    @functools.partial(
        pl.kernel,
        out_type=(
            jax.ShapeDtypeStruct((_NPAD, _D), jnp.float32),
            jax.ShapeDtypeStruct((_NPAD, 128), jnp.float32),
        ),
        mesh=mesh,
        scratch_types=scratch + [
            pltpu.VMEM((_CH, 128), jnp.float32),
            pltpu.SemaphoreType.DMA,
        ],
    )
    def sc_scatter(x_hbm, g_hbm, p_hbm, xs_hbm, gs_hbm, idx_v, rows_v, sem,
                   g_v, sem2):
        wid = lax.axis_index("s") * 2 + lax.axis_index("c")
        base = wid * _RPW
        pltpu.sync_copy(p_hbm.at[wid], idx_v)
        for c in range(_NCH):
            pltpu.sync_copy(x_hbm.at[pl.ds(base + c * _CH, _CH)], rows_v)
            pltpu.sync_copy(g_hbm.at[pl.ds(base + c * _CH, _CH)], g_v)
            cp1 = pltpu.async_copy(rows_v, xs_hbm.at[idx_v.at[c]], sem)
            cp2 = pltpu.async_copy(g_v, gs_hbm.at[idx_v.at[c]], sem2)
            cp1.wait()
            cp2.wait()

    @functools.partial(
        pl.kernel,
        out_type=jax.ShapeDtypeStruct((_N, _D), jnp.float32),
        mesh=mesh,
        scratch_types=scratch,
    )
    def sc_gather(os_hbm, p_hbm, y_hbm, idx_v, rows_v, sem):
        wid = lax.axis_index("s") * 2 + lax.axis_index("c")
        base = wid * _RPW
        pltpu.sync_copy(p_hbm.at[wid], idx_v)
        for c in range(_NCH):
            pltpu.async_copy(os_hbm.at[idx_v.at[c]], rows_v, sem).wait()
            pltpu.sync_copy(rows_v, y_hbm.at[pl.ds(base + c * _CH, _CH)])

    return sc_scatter, sc_gather


def kernel(x, token_mask, router_w, w1, w2):
    Bc, Tc, Dc = x.shape
    xf = x.reshape(_N, _D)
    maskf = token_mask.reshape(_N, 1).astype(jnp.float32)

    route_plan_call = pl.pallas_call(
        _route_plan_body,
        grid=(2 * _NB,),
        in_specs=[
            pl.BlockSpec((_TM, _D), lambda i: (jnp.minimum(i, _NB - 1), 0)),
            pl.BlockSpec((_E, _D), lambda i: (0, 0)),
            pl.BlockSpec((_TM, 1), lambda i: (jnp.minimum(i, _NB - 1), 0)),
        ],
        out_specs=[
            pl.BlockSpec((_TM, 128), lambda i: (jnp.minimum(i, _NB - 1), 0)),
            pl.BlockSpec((_TM, 1), lambda i: (jnp.maximum(i - _NB, 0), 0)),
            pl.BlockSpec((1, 128), lambda i: (0, 0)),
            pl.BlockSpec((1, 128), lambda i: (0, 0)),
        ],
        out_shape=[
            jax.ShapeDtypeStruct((_N, 128), jnp.float32),
            jax.ShapeDtypeStruct((_N, 1), jnp.int32),
            jax.ShapeDtypeStruct((1, 128), jnp.int32),
            jax.ShapeDtypeStruct((1, 128), jnp.int32),
        ],
        scratch_shapes=[
            pltpu.VMEM((_N, 1), jnp.float32),
            pltpu.VMEM((_N, 1), jnp.float32),
            pltpu.VMEM((1, _E), jnp.float32),
            pltpu.VMEM((1, _E), jnp.float32),
        ],
    )
    g16, p, te_pad, act_pad = route_plan_call(xf, router_w, maskf)

    p3 = p.reshape(_NW, _NCH, _CH)
    te = te_pad.reshape(128)[:_NT]
    act = act_pad.reshape(128)[:_NT]

    sc_scatter, sc_gather = _sc_kernels()
    xs, gs = sc_scatter(xf, g16, p3)

    ffn_call = pl.pallas_call(
        _ffn_body,
        grid_spec=pltpu.PrefetchScalarGridSpec(
            num_scalar_prefetch=2,
            grid=(_NT,),
            in_specs=[
                pl.BlockSpec((_TM, _D), lambda t, te_r, act_r: (t, 0)),
                pl.BlockSpec((_TM, 128), lambda t, te_r, act_r: (t, 0)),
                pl.BlockSpec((1, _D, _F), lambda t, te_r, act_r: (te_r[t], 0, 0)),
                pl.BlockSpec((1, _F, _D), lambda t, te_r, act_r: (te_r[t], 0, 0)),
            ],
            out_specs=pl.BlockSpec((_TM, _D), lambda t, te_r, act_r: (t, 0)),
            scratch_shapes=[
                pltpu.VMEM((_D, _F), jnp.bfloat16),
                pltpu.VMEM((_F, _D), jnp.bfloat16),
                pltpu.SMEM((1,), jnp.int32),
            ],
        ),
        out_shape=jax.ShapeDtypeStruct((_NPAD, _D), jnp.float32),
        compiler_params=pltpu.CompilerParams(
            dimension_semantics=("arbitrary",)),
    )
    osrt = ffn_call(te, act, xs, gs, w1, w2)

    y = sc_gather(osrt, p3)
    return y.reshape(Bc, Tc, Dc)


# speedup vs baseline: 1.1541x; 1.1170x over previous
import functools

import jax
import jax.numpy as jnp
from jax import lax
from jax.experimental import pallas as pl
from jax.experimental.pallas import tpu as pltpu
from jax.experimental.pallas import tpu_sc as plsc

_N = 4096
_D = 1024
_E = 8
_F = 2048
_TM = 256
_NB = _N // _TM
_NT = 23
_NPAD = _NT * _TM
_FB = 512
_NF = _F // _FB
_NW = 32
_RPW = _N // _NW
_CH = 64
_NCH = _RPW // _CH


def _route_plan_body(x_ref, rw_ref, mask_ref, g_ref, p_ref, te_ref, act_ref,
                     st_ref, rid_ref, ne_ref, nv_ref, e_all, r_all, acc, offs_s):
    i = pl.program_id(0)

    @pl.when(i == 0)
    def _():
        acc[...] = jnp.zeros_like(acc)

    lane = lax.broadcasted_iota(jnp.int32, (_TM, _E), 1).astype(jnp.float32)

    @pl.when(i < _NB)
    def _():
        xb = x_ref[...]
        logits = lax.dot_general(xb, rw_ref[...], (((1,), (1,)), ((), ())),
                                 preferred_element_type=jnp.float32)
        m = jnp.max(logits, axis=-1, keepdims=True)
        ex = jnp.exp(logits - m)
        s = jnp.sum(ex, axis=-1, keepdims=True)
        probs = ex / s
        pm = jnp.max(probs, axis=-1, keepdims=True)
        cand = jnp.where(probs >= pm, lane, float(_E))
        e = jnp.min(cand, axis=-1, keepdims=True)
        oh = (lane == e).astype(jnp.float32)
        r0 = lax.broadcasted_iota(jnp.int32, (_TM, _TM), 0)
        c0 = lax.broadcasted_iota(jnp.int32, (_TM, _TM), 1)
        tri = (c0 < r0).astype(jnp.float32)
        rank_all = jnp.dot(tri, oh, preferred_element_type=jnp.float32)
        base = acc[...]
        rank_tok = jnp.sum((rank_all + base) * oh, axis=-1, keepdims=True)
        gate = pm * mask_ref[...]
        g_ref[...] = jnp.broadcast_to(gate, (_TM, 128))
        e_all[pl.ds(i * _TM, _TM), :] = e
        r_all[pl.ds(i * _TM, _TM), :] = rank_tok
        acc[...] = base + jnp.sum(oh, axis=0, keepdims=True)

    @pl.when(i == _NB)
    def _():
        c = acc[...]
        a = jnp.ceil(c * (1.0 / _TM)) * float(_TM)
        r0 = lax.broadcasted_iota(jnp.int32, (_E, _E), 0)
        c0 = lax.broadcasted_iota(jnp.int32, (_E, _E), 1)
        triu = (r0 < c0).astype(jnp.float32)
        offs = jnp.dot(a, triu, preferred_element_type=jnp.float32)
        offs_s[...] = offs
        ends = offs + a
        eye = (r0 == c0).astype(jnp.float32)
        ends_col = lax.dot_general(eye, ends, (((1,), (1,)), ((), ())),
                                   preferred_element_type=jnp.float32)
        tt = lax.broadcasted_iota(jnp.int32, (_E, 128), 1).astype(jnp.float32) * float(_TM)
        te = jnp.sum((ends_col <= tt).astype(jnp.float32), axis=0, keepdims=True)
        te_ref[...] = jnp.minimum(te, float(_E - 1)).astype(jnp.int32)
        total = jnp.sum(a, axis=-1, keepdims=True)
        trow = lax.broadcasted_iota(jnp.int32, (1, 128), 1).astype(jnp.float32) * float(_TM)
        act_ref[...] = (trow < total).astype(jnp.int32)
        offs_col = lax.dot_general(eye, offs, (((1,), (1,)), ((), ())),
                                   preferred_element_type=jnp.float32)
        a_col = ends_col - offs_col
        has = (a_col > 0.0).astype(jnp.float32)
        st = jnp.sum(((offs_col == tt).astype(jnp.float32)) * has,
                     axis=0, keepdims=True)
        st_ref[...] = st.astype(jnp.int32)
        rid = jnp.sum(((offs_col <= tt).astype(jnp.float32)) * has,
                      axis=0, keepdims=True) - 1.0
        rid_ref[...] = jnp.maximum(rid, 0.0).astype(jnp.int32)
        e_col = lax.broadcasted_iota(jnp.int32, (_E, 128), 0).astype(jnp.float32)
        cand_n = jnp.where((a_col > 0.0) & (offs_col > tt), e_col, float(_E))
        ne = jnp.min(cand_n, axis=0, keepdims=True)
        nv_ref[...] = (ne < float(_E)).astype(jnp.int32)
        ne_ref[...] = jnp.minimum(ne, float(_E - 1)).astype(jnp.int32)

    @pl.when(i >= _NB)
    def _():
        j = i - _NB
        e = e_all[pl.ds(j * _TM, _TM), :]
        oh = (lane == e).astype(jnp.float32)
        off_tok = jnp.sum(oh * offs_s[...], axis=-1, keepdims=True)
        p_ref[...] = (off_tok + r_all[pl.ds(j * _TM, _TM), :]).astype(jnp.int32)


def _ffn_body(te_ref, act_ref, st_ref, rid_ref, ne_ref, nv_ref,
              xs_ref, gs_ref, w1_hbm, w2_hbm, o_ref,
              slots1, slots2, w1c, w2c, sem1, sem2):
    t = pl.program_id(0)
    active = act_ref[t] == 1
    start = jnp.logical_and(active, st_ref[t] == 1)
    rid = rid_ref[t]
    sl = lax.rem(rid, 2)

    def _issue(e, s):
        pltpu.make_async_copy(w1_hbm.at[e], slots1.at[s], sem1.at[s]).start()
        pltpu.make_async_copy(w2_hbm.at[e], slots2.at[s], sem2.at[s]).start()

    @pl.when(t == 0)
    def _():
        _issue(te_ref[0], 0)

        @pl.when(nv_ref[0] == 1)
        def _():
            _issue(ne_ref[0], 1)

    @pl.when(start)
    def _():
        pltpu.make_async_copy(w1_hbm.at[te_ref[t]], slots1.at[sl], sem1.at[sl]).wait()
        pltpu.make_async_copy(w2_hbm.at[te_ref[t]], slots2.at[sl], sem2.at[sl]).wait()
        w1c[...] = slots1[sl].astype(jnp.bfloat16)
        w2c[...] = slots2[sl].astype(jnp.bfloat16)

    @pl.when(jnp.logical_and(start, jnp.logical_and(rid >= 1, nv_ref[t] == 1)))
    def _():
        _issue(ne_ref[t], 1 - sl)

    @pl.when(active)
    def _():
        xb16 = (xs_ref[...] * gs_ref[:, 0:1]).astype(jnp.bfloat16)
        h = jnp.dot(xb16, w1c[...], preferred_element_type=jnp.float32)
        h16 = jnp.maximum(h, 0.0).astype(jnp.bfloat16)
        o_ref[...] = jnp.dot(h16, w2c[...], preferred_element_type=jnp.float32)


@functools.cache
def _sc_kernels():
    mesh = plsc.VectorSubcoreMesh(core_axis_name="c", subcore_axis_name="s")
    scratch = [
        pltpu.VMEM((_NCH, _CH), jnp.int32),
        pltpu.VMEM((_CH, _D), jnp.float32),
        pltpu.SemaphoreType.DMA,
    ]

    @functools.partial(
        pl.kernel,
        out_type=(
            jax.ShapeDtypeStruct((_NPAD, _D), jnp.float32),
            jax.ShapeDtypeStruct((_NPAD, 128), jnp.float32),
        ),
        mesh=mesh,
        scratch_types=scratch + [
            pltpu.VMEM((_CH, 128), jnp.float32),
            pltpu.SemaphoreType.DMA,
        ],
    )
    def sc_scatter(x_hbm, g_hbm, p_hbm, xs_hbm, gs_hbm, idx_v, rows_v, sem,
                   g_v, sem2):
        wid = lax.axis_index("s") * 2 + lax.axis_index("c")
        base = wid * _RPW
        pltpu.sync_copy(p_hbm.at[wid], idx_v)
        for c in range(_NCH):
            pltpu.sync_copy(x_hbm.at[pl.ds(base + c * _CH, _CH)], rows_v)
            pltpu.sync_copy(g_hbm.at[pl.ds(base + c * _CH, _CH)], g_v)
            cp1 = pltpu.async_copy(rows_v, xs_hbm.at[idx_v.at[c]], sem)
            cp2 = pltpu.async_copy(g_v, gs_hbm.at[idx_v.at[c]], sem2)
            cp1.wait()
            cp2.wait()

    @functools.partial(
        pl.kernel,
        out_type=jax.ShapeDtypeStruct((_N, _D), jnp.float32),
        mesh=mesh,
        scratch_types=scratch,
    )
    def sc_gather(os_hbm, p_hbm, y_hbm, idx_v, rows_v, sem):
        wid = lax.axis_index("s") * 2 + lax.axis_index("c")
        base = wid * _RPW
        pltpu.sync_copy(p_hbm.at[wid], idx_v)
        for c in range(_NCH):
            pltpu.async_copy(os_hbm.at[idx_v.at[c]], rows_v, sem).wait()
            pltpu.sync_copy(rows_v, y_hbm.at[pl.ds(base + c * _CH, _CH)])

    return sc_scatter, sc_gather


def kernel(x, token_mask, router_w, w1, w2):
    Bc, Tc, Dc = x.shape
    xf = x.reshape(_N, _D)
    maskf = token_mask.reshape(_N, 1).astype(jnp.float32)

    route_plan_call = pl.pallas_call(
        _route_plan_body,
        grid=(2 * _NB,),
        in_specs=[
            pl.BlockSpec((_TM, _D), lambda i: (jnp.minimum(i, _NB - 1), 0)),
            pl.BlockSpec((_E, _D), lambda i: (0, 0)),
            pl.BlockSpec((_TM, 1), lambda i: (jnp.minimum(i, _NB - 1), 0)),
        ],
        out_specs=[
            pl.BlockSpec((_TM, 128), lambda i: (jnp.minimum(i, _NB - 1), 0)),
            pl.BlockSpec((_TM, 1), lambda i: (jnp.maximum(i - _NB, 0), 0)),
        ] + [pl.BlockSpec((1, 128), lambda i: (0, 0))] * 6,
        out_shape=[
            jax.ShapeDtypeStruct((_N, 128), jnp.float32),
            jax.ShapeDtypeStruct((_N, 1), jnp.int32),
        ] + [jax.ShapeDtypeStruct((1, 128), jnp.int32)] * 6,
        scratch_shapes=[
            pltpu.VMEM((_N, 1), jnp.float32),
            pltpu.VMEM((_N, 1), jnp.float32),
            pltpu.VMEM((1, _E), jnp.float32),
            pltpu.VMEM((1, _E), jnp.float32),
        ],
    )
    g16, p, te_pad, act_pad, st_pad, rid_pad, ne_pad, nv_pad = route_plan_call(
        xf, router_w, maskf)

    p3 = p.reshape(_NW, _NCH, _CH)
    te = te_pad.reshape(128)[:_NT]
    act = act_pad.reshape(128)[:_NT]
    stt = st_pad.reshape(128)[:_NT]
    rid = rid_pad.reshape(128)[:_NT]
    ne = ne_pad.reshape(128)[:_NT]
    nv = nv_pad.reshape(128)[:_NT]

    sc_scatter, sc_gather = _sc_kernels()
    xs, gs = sc_scatter(xf, g16, p3)

    ffn_call = pl.pallas_call(
        _ffn_body,
        grid_spec=pltpu.PrefetchScalarGridSpec(
            num_scalar_prefetch=6,
            grid=(_NT,),
            in_specs=[
                pl.BlockSpec((_TM, _D), lambda t, *_: (t, 0)),
                pl.BlockSpec((_TM, 128), lambda t, *_: (t, 0)),
                pl.BlockSpec(memory_space=pl.ANY),
                pl.BlockSpec(memory_space=pl.ANY),
            ],
            out_specs=pl.BlockSpec((_TM, _D), lambda t, *_: (t, 0)),
            scratch_shapes=[
                pltpu.VMEM((2, _D, _F), jnp.float32),
                pltpu.VMEM((2, _F, _D), jnp.float32),
                pltpu.VMEM((_D, _F), jnp.bfloat16),
                pltpu.VMEM((_F, _D), jnp.bfloat16),
                pltpu.SemaphoreType.DMA((2,)),
                pltpu.SemaphoreType.DMA((2,)),
            ],
        ),
        out_shape=jax.ShapeDtypeStruct((_NPAD, _D), jnp.float32),
        compiler_params=pltpu.CompilerParams(
            dimension_semantics=("arbitrary",)),
    )
    osrt = ffn_call(te, act, stt, rid, ne, nv, xs, gs, w1, w2)

    y = sc_gather(osrt, p3)
    return y.reshape(Bc, Tc, Dc)


# speedup vs baseline: 1.2254x; 1.0618x over previous
import functools

import jax
import jax.numpy as jnp
from jax import lax
from jax.experimental import pallas as pl
from jax.experimental.pallas import tpu as pltpu
from jax.experimental.pallas import tpu_sc as plsc

_N = 4096
_D = 1024
_E = 8
_F = 2048
_TM = 256
_NB = _N // _TM
_TMR = 512
_NBR = _N // _TMR
_TMP = 1024
_NBP = _N // _TMP
_NT = 23
_NPAD = _NT * _TM
_FB = 512
_NF = _F // _FB
_NW = 32
_RPW = _N // _NW
_CH = 64
_NCH = _RPW // _CH


def _route_plan_body(x_ref, rw_ref, mask_ref, g_ref, p_ref, te_ref, act_ref,
                     st_ref, rid_ref, ne_ref, nv_ref, e_all, r_all, acc, offs_s):
    i = pl.program_id(0)

    @pl.when(i == 0)
    def _():
        acc[...] = jnp.zeros_like(acc)

    @pl.when(i < _NBR)
    def _():
        lane = lax.broadcasted_iota(jnp.int32, (_TMR, _E), 1).astype(jnp.float32)
        xb = x_ref[...]
        logits = lax.dot_general(xb, rw_ref[...], (((1,), (1,)), ((), ())),
                                 preferred_element_type=jnp.float32)
        m = jnp.max(logits, axis=-1, keepdims=True)
        ex = jnp.exp(logits - m)
        s = jnp.sum(ex, axis=-1, keepdims=True)
        probs = ex / s
        pm = jnp.max(probs, axis=-1, keepdims=True)
        cand = jnp.where(probs >= pm, lane, float(_E))
        e = jnp.min(cand, axis=-1, keepdims=True)
        oh = (lane == e).astype(jnp.float32)
        r0 = lax.broadcasted_iota(jnp.int32, (_TMR, _TMR), 0)
        c0 = lax.broadcasted_iota(jnp.int32, (_TMR, _TMR), 1)
        tri = (c0 < r0).astype(jnp.float32)
        rank_all = jnp.dot(tri, oh, preferred_element_type=jnp.float32)
        base = acc[...]
        rank_tok = jnp.sum((rank_all + base) * oh, axis=-1, keepdims=True)
        gate = pm * mask_ref[...]
        g_ref[...] = jnp.broadcast_to(gate, (_TMR, 128))
        e_all[pl.ds(i * _TMR, _TMR), :] = e
        r_all[pl.ds(i * _TMR, _TMR), :] = rank_tok
        acc[...] = base + jnp.sum(oh, axis=0, keepdims=True)

    @pl.when(i == _NBR)
    def _():
        c = acc[...]
        a = jnp.ceil(c * (1.0 / _TM)) * float(_TM)
        r0 = lax.broadcasted_iota(jnp.int32, (_E, _E), 0)
        c0 = lax.broadcasted_iota(jnp.int32, (_E, _E), 1)
        triu = (r0 < c0).astype(jnp.float32)
        offs = jnp.dot(a, triu, preferred_element_type=jnp.float32)
        offs_s[...] = offs
        ends = offs + a
        eye = (r0 == c0).astype(jnp.float32)
        ends_col = lax.dot_general(eye, ends, (((1,), (1,)), ((), ())),
                                   preferred_element_type=jnp.float32)
        tt = lax.broadcasted_iota(jnp.int32, (_E, 128), 1).astype(jnp.float32) * float(_TM)
        te = jnp.sum((ends_col <= tt).astype(jnp.float32), axis=0, keepdims=True)
        te_ref[...] = jnp.minimum(te, float(_E - 1)).astype(jnp.int32)
        total = jnp.sum(a, axis=-1, keepdims=True)
        trow = lax.broadcasted_iota(jnp.int32, (1, 128), 1).astype(jnp.float32) * float(_TM)
        act_ref[...] = (trow < total).astype(jnp.int32)
        offs_col = lax.dot_general(eye, offs, (((1,), (1,)), ((), ())),
                                   preferred_element_type=jnp.float32)
        a_col = ends_col - offs_col
        has = (a_col > 0.0).astype(jnp.float32)
        st = jnp.sum(((offs_col == tt).astype(jnp.float32)) * has,
                     axis=0, keepdims=True)
        st_ref[...] = st.astype(jnp.int32)
        rid = jnp.sum(((offs_col <= tt).astype(jnp.float32)) * has,
                      axis=0, keepdims=True) - 1.0
        rid_ref[...] = jnp.maximum(rid, 0.0).astype(jnp.int32)
        e_col = lax.broadcasted_iota(jnp.int32, (_E, 128), 0).astype(jnp.float32)
        cand_n = jnp.where((a_col > 0.0) & (offs_col > tt), e_col, float(_E))
        ne = jnp.min(cand_n, axis=0, keepdims=True)
        nv_ref[...] = (ne < float(_E)).astype(jnp.int32)
        ne_ref[...] = jnp.minimum(ne, float(_E - 1)).astype(jnp.int32)

    @pl.when(i >= _NBR)
    def _():
        j = i - _NBR
        lane = lax.broadcasted_iota(jnp.int32, (_TMP, _E), 1).astype(jnp.float32)
        e = e_all[pl.ds(j * _TMP, _TMP), :]
        oh = (lane == e).astype(jnp.float32)
        off_tok = jnp.sum(oh * offs_s[...], axis=-1, keepdims=True)
        p_ref[...] = (off_tok + r_all[pl.ds(j * _TMP, _TMP), :]).astype(jnp.int32)


def _ffn_body(te_ref, act_ref, st_ref, rid_ref, ne_ref, nv_ref,
              xs_ref, gs_ref, w1_hbm, w2_hbm, o_ref,
              slots1, slots2, w1c, w2c, sem1, sem2):
    t = pl.program_id(0)
    active = act_ref[t] == 1
    start = jnp.logical_and(active, st_ref[t] == 1)
    rid = rid_ref[t]
    sl = lax.rem(rid, 2)

    def _issue(e, s):
        pltpu.make_async_copy(w1_hbm.at[e], slots1.at[s], sem1.at[s]).start()
        pltpu.make_async_copy(w2_hbm.at[e], slots2.at[s], sem2.at[s]).start()

    @pl.when(t == 0)
    def _():
        _issue(te_ref[0], 0)

        @pl.when(nv_ref[0] == 1)
        def _():
            _issue(ne_ref[0], 1)

    @pl.when(start)
    def _():
        pltpu.make_async_copy(w1_hbm.at[te_ref[t]], slots1.at[sl], sem1.at[sl]).wait()
        pltpu.make_async_copy(w2_hbm.at[te_ref[t]], slots2.at[sl], sem2.at[sl]).wait()
        w1c[...] = slots1[sl].astype(jnp.bfloat16)
        w2c[...] = slots2[sl].astype(jnp.bfloat16)

    @pl.when(jnp.logical_and(start, jnp.logical_and(rid >= 1, nv_ref[t] == 1)))
    def _():
        _issue(ne_ref[t], 1 - sl)

    @pl.when(active)
    def _():
        xb16 = (xs_ref[...] * gs_ref[:, 0:1]).astype(jnp.bfloat16)
        h = jnp.dot(xb16, w1c[...], preferred_element_type=jnp.float32)
        h16 = jnp.maximum(h, 0.0).astype(jnp.bfloat16)
        o_ref[...] = jnp.dot(h16, w2c[...], preferred_element_type=jnp.float32)


@functools.cache
def _sc_kernels():
    mesh = plsc.VectorSubcoreMesh(core_axis_name="c", subcore_axis_name="s")
    scratch = [
        pltpu.VMEM((_NCH, _CH), jnp.int32),
        pltpu.VMEM((_CH, _D), jnp.float32),
        pltpu.SemaphoreType.DMA,
    ]

    @functools.partial(
        pl.kernel,
        out_type=(
            jax.ShapeDtypeStruct((_NPAD, _D), jnp.float32),
            jax.ShapeDtypeStruct((_NPAD, 128), jnp.float32),
        ),
        mesh=mesh,
        scratch_types=scratch + [
            pltpu.VMEM((_CH, 128), jnp.float32),
            pltpu.SemaphoreType.DMA,
        ],
    )
    def sc_scatter(x_hbm, g_hbm, p_hbm, xs_hbm, gs_hbm, idx_v, rows_v, sem,
                   g_v, sem2):
        wid = lax.axis_index("s") * 2 + lax.axis_index("c")
        base = wid * _RPW
        pltpu.sync_copy(p_hbm.at[wid], idx_v)
        for c in range(_NCH):
            pltpu.sync_copy(x_hbm.at[pl.ds(base + c * _CH, _CH)], rows_v)
            pltpu.sync_copy(g_hbm.at[pl.ds(base + c * _CH, _CH)], g_v)
            cp1 = pltpu.async_copy(rows_v, xs_hbm.at[idx_v.at[c]], sem)
            cp2 = pltpu.async_copy(g_v, gs_hbm.at[idx_v.at[c]], sem2)
            cp1.wait()
            cp2.wait()

    @functools.partial(
        pl.kernel,
        out_type=jax.ShapeDtypeStruct((_N, _D), jnp.float32),
        mesh=mesh,
        scratch_types=scratch,
    )
    def sc_gather(os_hbm, p_hbm, y_hbm, idx_v, rows_v, sem):
        wid = lax.axis_index("s") * 2 + lax.axis_index("c")
        base = wid * _RPW
        pltpu.sync_copy(p_hbm.at[wid], idx_v)
        for c in range(_NCH):
            pltpu.async_copy(os_hbm.at[idx_v.at[c]], rows_v, sem).wait()
            pltpu.sync_copy(rows_v, y_hbm.at[pl.ds(base + c * _CH, _CH)])

    return sc_scatter, sc_gather


def kernel(x, token_mask, router_w, w1, w2):
    Bc, Tc, Dc = x.shape
    xf = x.reshape(_N, _D)
    maskf = token_mask.reshape(_N, 1).astype(jnp.float32)

    route_plan_call = pl.pallas_call(
        _route_plan_body,
        grid=(_NBR + _NBP,),
        in_specs=[
            pl.BlockSpec((_TMR, _D), lambda i: (jnp.minimum(i, _NBR - 1), 0)),
            pl.BlockSpec((_E, _D), lambda i: (0, 0)),
            pl.BlockSpec((_TMR, 1), lambda i: (jnp.minimum(i, _NBR - 1), 0)),
        ],
        out_specs=[
            pl.BlockSpec((_TMR, 128), lambda i: (jnp.minimum(i, _NBR - 1), 0)),
            pl.BlockSpec((_TMP, 1), lambda i: (jnp.maximum(i - _NBR, 0), 0)),
        ] + [pl.BlockSpec((1, 128), lambda i: (0, 0))] * 6,
        out_shape=[
            jax.ShapeDtypeStruct((_N, 128), jnp.float32),
            jax.ShapeDtypeStruct((_N, 1), jnp.int32),
        ] + [jax.ShapeDtypeStruct((1, 128), jnp.int32)] * 6,
        scratch_shapes=[
            pltpu.VMEM((_N, 1), jnp.float32),
            pltpu.VMEM((_N, 1), jnp.float32),
            pltpu.VMEM((1, _E), jnp.float32),
            pltpu.VMEM((1, _E), jnp.float32),
        ],
    )
    g16, p, te_pad, act_pad, st_pad, rid_pad, ne_pad, nv_pad = route_plan_call(
        xf, router_w, maskf)

    p3 = p.reshape(_NW, _NCH, _CH)
    te = te_pad.reshape(128)[:_NT]
    act = act_pad.reshape(128)[:_NT]
    stt = st_pad.reshape(128)[:_NT]
    rid = rid_pad.reshape(128)[:_NT]
    ne = ne_pad.reshape(128)[:_NT]
    nv = nv_pad.reshape(128)[:_NT]

    sc_scatter, sc_gather = _sc_kernels()
    xs, gs = sc_scatter(xf, g16, p3)

    ffn_call = pl.pallas_call(
        _ffn_body,
        grid_spec=pltpu.PrefetchScalarGridSpec(
            num_scalar_prefetch=6,
            grid=(_NT,),
            in_specs=[
                pl.BlockSpec((_TM, _D), lambda t, *_: (t, 0)),
                pl.BlockSpec((_TM, 128), lambda t, *_: (t, 0)),
                pl.BlockSpec(memory_space=pl.ANY),
                pl.BlockSpec(memory_space=pl.ANY),
            ],
            out_specs=pl.BlockSpec((_TM, _D), lambda t, *_: (t, 0)),
            scratch_shapes=[
                pltpu.VMEM((2, _D, _F), jnp.float32),
                pltpu.VMEM((2, _F, _D), jnp.float32),
                pltpu.VMEM((_D, _F), jnp.bfloat16),
                pltpu.VMEM((_F, _D), jnp.bfloat16),
                pltpu.SemaphoreType.DMA((2,)),
                pltpu.SemaphoreType.DMA((2,)),
            ],
        ),
        out_shape=jax.ShapeDtypeStruct((_NPAD, _D), jnp.float32),
        compiler_params=pltpu.CompilerParams(
            dimension_semantics=("arbitrary",)),
    )
    osrt = ffn_call(te, act, stt, rid, ne, nv, xs, gs, w1, w2)

    y = sc_gather(osrt, p3)
    return y.reshape(Bc, Tc, Dc)


# speedup vs baseline: 1.2468x; 1.0175x over previous
import functools

import jax
import jax.numpy as jnp
from jax import lax
from jax.experimental import pallas as pl
from jax.experimental.pallas import tpu as pltpu
from jax.experimental.pallas import tpu_sc as plsc

_N = 4096
_D = 1024
_E = 8
_F = 2048
_TM = 256
_NB = _N // _TM
_TMR = 512
_NBR = _N // _TMR
_TMP = 1024
_NBP = _N // _TMP
_NT = 23
_NPAD = _NT * _TM
_FB = 512
_NF = _F // _FB
_NW = 32
_RPW = _N // _NW
_CH = 32
_NCH = _RPW // _CH


def _route_plan_body(x_ref, rw_ref, mask_ref, g_ref, p_ref, te_ref, act_ref,
                     st_ref, rid_ref, ne_ref, nv_ref, e_all, r_all, acc, offs_s):
    i = pl.program_id(0)

    @pl.when(i == 0)
    def _():
        acc[...] = jnp.zeros_like(acc)

    @pl.when(i < _NBR)
    def _():
        lane = lax.broadcasted_iota(jnp.int32, (_TMR, _E), 1).astype(jnp.float32)
        xb = x_ref[...]
        logits = lax.dot_general(xb, rw_ref[...], (((1,), (1,)), ((), ())),
                                 preferred_element_type=jnp.float32)
        m = jnp.max(logits, axis=-1, keepdims=True)
        ex = jnp.exp(logits - m)
        s = jnp.sum(ex, axis=-1, keepdims=True)
        probs = ex / s
        pm = jnp.max(probs, axis=-1, keepdims=True)
        cand = jnp.where(probs >= pm, lane, float(_E))
        e = jnp.min(cand, axis=-1, keepdims=True)
        oh = (lane == e).astype(jnp.float32)
        r0 = lax.broadcasted_iota(jnp.int32, (_TMR, _TMR), 0)
        c0 = lax.broadcasted_iota(jnp.int32, (_TMR, _TMR), 1)
        tri = (c0 < r0).astype(jnp.float32)
        rank_all = jnp.dot(tri, oh, preferred_element_type=jnp.float32)
        base = acc[...]
        rank_tok = jnp.sum((rank_all + base) * oh, axis=-1, keepdims=True)
        gate = pm * mask_ref[...]
        g_ref[...] = jnp.broadcast_to(gate, (_TMR, 128))
        e_all[pl.ds(i * _TMR, _TMR), :] = e
        r_all[pl.ds(i * _TMR, _TMR), :] = rank_tok
        acc[...] = base + jnp.sum(oh, axis=0, keepdims=True)

    @pl.when(i == _NBR)
    def _():
        c = acc[...]
        a = jnp.ceil(c * (1.0 / _TM)) * float(_TM)
        r0 = lax.broadcasted_iota(jnp.int32, (_E, _E), 0)
        c0 = lax.broadcasted_iota(jnp.int32, (_E, _E), 1)
        triu = (r0 < c0).astype(jnp.float32)
        offs = jnp.dot(a, triu, preferred_element_type=jnp.float32)
        offs_s[...] = offs
        ends = offs + a
        eye = (r0 == c0).astype(jnp.float32)
        ends_col = lax.dot_general(eye, ends, (((1,), (1,)), ((), ())),
                                   preferred_element_type=jnp.float32)
        tt = lax.broadcasted_iota(jnp.int32, (_E, 128), 1).astype(jnp.float32) * float(_TM)
        te = jnp.sum((ends_col <= tt).astype(jnp.float32), axis=0, keepdims=True)
        te_ref[...] = jnp.minimum(te, float(_E - 1)).astype(jnp.int32)
        total = jnp.sum(a, axis=-1, keepdims=True)
        trow = lax.broadcasted_iota(jnp.int32, (1, 128), 1).astype(jnp.float32) * float(_TM)
        act_ref[...] = (trow < total).astype(jnp.int32)
        offs_col = lax.dot_general(eye, offs, (((1,), (1,)), ((), ())),
                                   preferred_element_type=jnp.float32)
        a_col = ends_col - offs_col
        has = (a_col > 0.0).astype(jnp.float32)
        st = jnp.sum(((offs_col == tt).astype(jnp.float32)) * has,
                     axis=0, keepdims=True)
        st_ref[...] = st.astype(jnp.int32)
        rid = jnp.sum(((offs_col <= tt).astype(jnp.float32)) * has,
                      axis=0, keepdims=True) - 1.0
        rid_ref[...] = jnp.maximum(rid, 0.0).astype(jnp.int32)
        e_col = lax.broadcasted_iota(jnp.int32, (_E, 128), 0).astype(jnp.float32)
        cand_n = jnp.where((a_col > 0.0) & (offs_col > tt), e_col, float(_E))
        ne = jnp.min(cand_n, axis=0, keepdims=True)
        nv_ref[...] = (ne < float(_E)).astype(jnp.int32)
        ne_ref[...] = jnp.minimum(ne, float(_E - 1)).astype(jnp.int32)

    @pl.when(i >= _NBR)
    def _():
        j = i - _NBR
        lane = lax.broadcasted_iota(jnp.int32, (_TMP, _E), 1).astype(jnp.float32)
        e = e_all[pl.ds(j * _TMP, _TMP), :]
        oh = (lane == e).astype(jnp.float32)
        off_tok = jnp.sum(oh * offs_s[...], axis=-1, keepdims=True)
        p_ref[...] = (off_tok + r_all[pl.ds(j * _TMP, _TMP), :]).astype(jnp.int32)


def _ffn_body(te_ref, act_ref, st_ref, rid_ref, ne_ref, nv_ref,
              xs_ref, gs_ref, w1_hbm, w2_hbm, o_ref,
              slots1, slots2, w1c, w2c, sem1, sem2):
    t = pl.program_id(0)
    active = act_ref[t] == 1
    start = jnp.logical_and(active, st_ref[t] == 1)
    rid = rid_ref[t]
    sl = lax.rem(rid, 2)

    def _issue(e, s):
        pltpu.make_async_copy(w1_hbm.at[e], slots1.at[s], sem1.at[s]).start()
        pltpu.make_async_copy(w2_hbm.at[e], slots2.at[s], sem2.at[s]).start()

    @pl.when(t == 0)
    def _():
        _issue(te_ref[0], 0)

        @pl.when(nv_ref[0] == 1)
        def _():
            _issue(ne_ref[0], 1)

    @pl.when(start)
    def _():
        pltpu.make_async_copy(w1_hbm.at[te_ref[t]], slots1.at[sl], sem1.at[sl]).wait()
        pltpu.make_async_copy(w2_hbm.at[te_ref[t]], slots2.at[sl], sem2.at[sl]).wait()
        w1c[...] = slots1[sl].astype(jnp.bfloat16)
        w2c[...] = slots2[sl].astype(jnp.bfloat16)

    @pl.when(jnp.logical_and(start, jnp.logical_and(rid >= 1, nv_ref[t] == 1)))
    def _():
        _issue(ne_ref[t], 1 - sl)

    @pl.when(active)
    def _():
        xb16 = (xs_ref[...] * gs_ref[:, 0:1]).astype(jnp.bfloat16)
        h = jnp.dot(xb16, w1c[...], preferred_element_type=jnp.float32)
        h16 = jnp.maximum(h, 0.0).astype(jnp.bfloat16)
        o_ref[...] = jnp.dot(h16, w2c[...], preferred_element_type=jnp.float32)


@functools.cache
def _sc_kernels():
    mesh = plsc.VectorSubcoreMesh(core_axis_name="c", subcore_axis_name="s")

    @functools.partial(
        pl.kernel,
        out_type=(
            jax.ShapeDtypeStruct((_NPAD, _D), jnp.float32),
            jax.ShapeDtypeStruct((_NPAD, 128), jnp.float32),
        ),
        mesh=mesh,
        scratch_types=[
            pltpu.VMEM((_NCH, _CH), jnp.int32),
            pltpu.VMEM((3, _CH, _D), jnp.float32),
            pltpu.VMEM((3, _CH, 128), jnp.float32),
            pltpu.SemaphoreType.DMA((3,)),
            pltpu.SemaphoreType.DMA((3,)),
            pltpu.SemaphoreType.DMA((3,)),
            pltpu.SemaphoreType.DMA((3,)),
        ],
    )
    def sc_scatter(x_hbm, g_hbm, p_hbm, xs_hbm, gs_hbm,
                   idx_v, rows, gbuf, semL, semG, semSL, semSG):
        wid = lax.axis_index("s") * 2 + lax.axis_index("c")
        base = wid * _RPW
        pltpu.sync_copy(p_hbm.at[wid], idx_v)

        def load(c):
            s = c % 3
            pltpu.make_async_copy(x_hbm.at[pl.ds(base + c * _CH, _CH)],
                                  rows.at[s], semL.at[s]).start()
            pltpu.make_async_copy(g_hbm.at[pl.ds(base + c * _CH, _CH)],
                                  gbuf.at[s], semG.at[s]).start()

        def wait_load(c):
            s = c % 3
            pltpu.make_async_copy(x_hbm.at[pl.ds(base, _CH)],
                                  rows.at[s], semL.at[s]).wait()
            pltpu.make_async_copy(g_hbm.at[pl.ds(base, _CH)],
                                  gbuf.at[s], semG.at[s]).wait()

        def store(c):
            s = c % 3
            pltpu.make_async_copy(rows.at[s], xs_hbm.at[idx_v.at[c]],
                                  semSL.at[s]).start()
            pltpu.make_async_copy(gbuf.at[s], gs_hbm.at[idx_v.at[c]],
                                  semSG.at[s]).start()

        def wait_store(c):
            s = c % 3
            pltpu.make_async_copy(rows.at[s], xs_hbm.at[idx_v.at[c]],
                                  semSL.at[s]).wait()
            pltpu.make_async_copy(gbuf.at[s], gs_hbm.at[idx_v.at[c]],
                                  semSG.at[s]).wait()

        for c in range(min(3, _NCH)):
            load(c)
        for c in range(_NCH):
            wait_load(c)
            store(c)
            if c + 3 < _NCH:
                wait_store(c)
                load(c + 3)
        for c in range(max(0, _NCH - 3), _NCH):
            wait_store(c)

    @functools.partial(
        pl.kernel,
        out_type=jax.ShapeDtypeStruct((_N, _D), jnp.float32),
        mesh=mesh,
        scratch_types=[
            pltpu.VMEM((_NCH, _CH), jnp.int32),
            pltpu.VMEM((3, _CH, _D), jnp.float32),
            pltpu.SemaphoreType.DMA((3,)),
            pltpu.SemaphoreType.DMA((3,)),
        ],
    )
    def sc_gather(os_hbm, p_hbm, y_hbm, idx_v, rows, semL, semS):
        wid = lax.axis_index("s") * 2 + lax.axis_index("c")
        base = wid * _RPW
        pltpu.sync_copy(p_hbm.at[wid], idx_v)

        def load(c):
            s = c % 3
            pltpu.make_async_copy(os_hbm.at[idx_v.at[c]], rows.at[s],
                                  semL.at[s]).start()

        def wait_load(c):
            s = c % 3
            pltpu.make_async_copy(os_hbm.at[idx_v.at[c]], rows.at[s],
                                  semL.at[s]).wait()

        def store(c):
            s = c % 3
            pltpu.make_async_copy(rows.at[s],
                                  y_hbm.at[pl.ds(base + c * _CH, _CH)],
                                  semS.at[s]).start()

        def wait_store(c):
            s = c % 3
            pltpu.make_async_copy(rows.at[s],
                                  y_hbm.at[pl.ds(base + c * _CH, _CH)],
                                  semS.at[s]).wait()

        for c in range(min(3, _NCH)):
            load(c)
        for c in range(_NCH):
            wait_load(c)
            store(c)
            if c + 3 < _NCH:
                wait_store(c)
                load(c + 3)
        for c in range(max(0, _NCH - 3), _NCH):
            wait_store(c)

    return sc_scatter, sc_gather


def kernel(x, token_mask, router_w, w1, w2):
    Bc, Tc, Dc = x.shape
    xf = x.reshape(_N, _D)
    maskf = token_mask.reshape(_N, 1).astype(jnp.float32)

    route_plan_call = pl.pallas_call(
        _route_plan_body,
        grid=(_NBR + _NBP,),
        in_specs=[
            pl.BlockSpec((_TMR, _D), lambda i: (jnp.minimum(i, _NBR - 1), 0)),
            pl.BlockSpec((_E, _D), lambda i: (0, 0)),
            pl.BlockSpec((_TMR, 1), lambda i: (jnp.minimum(i, _NBR - 1), 0)),
        ],
        out_specs=[
            pl.BlockSpec((_TMR, 128), lambda i: (jnp.minimum(i, _NBR - 1), 0)),
            pl.BlockSpec((_TMP, 1), lambda i: (jnp.maximum(i - _NBR, 0), 0)),
        ] + [pl.BlockSpec((1, 128), lambda i: (0, 0))] * 6,
        out_shape=[
            jax.ShapeDtypeStruct((_N, 128), jnp.float32),
            jax.ShapeDtypeStruct((_N, 1), jnp.int32),
        ] + [jax.ShapeDtypeStruct((1, 128), jnp.int32)] * 6,
        scratch_shapes=[
            pltpu.VMEM((_N, 1), jnp.float32),
            pltpu.VMEM((_N, 1), jnp.float32),
            pltpu.VMEM((1, _E), jnp.float32),
            pltpu.VMEM((1, _E), jnp.float32),
        ],
    )
    g16, p, te_pad, act_pad, st_pad, rid_pad, ne_pad, nv_pad = route_plan_call(
        xf, router_w, maskf)

    p3 = p.reshape(_NW, _NCH, _CH)
    te = te_pad.reshape(128)[:_NT]
    act = act_pad.reshape(128)[:_NT]
    stt = st_pad.reshape(128)[:_NT]
    rid = rid_pad.reshape(128)[:_NT]
    ne = ne_pad.reshape(128)[:_NT]
    nv = nv_pad.reshape(128)[:_NT]

    sc_scatter, sc_gather = _sc_kernels()
    xs, gs = sc_scatter(xf, g16, p3)

    ffn_call = pl.pallas_call(
        _ffn_body,
        grid_spec=pltpu.PrefetchScalarGridSpec(
            num_scalar_prefetch=6,
            grid=(_NT,),
            in_specs=[
                pl.BlockSpec((_TM, _D), lambda t, *_: (t, 0)),
                pl.BlockSpec((_TM, 128), lambda t, *_: (t, 0)),
                pl.BlockSpec(memory_space=pl.ANY),
                pl.BlockSpec(memory_space=pl.ANY),
            ],
            out_specs=pl.BlockSpec((_TM, _D), lambda t, *_: (t, 0)),
            scratch_shapes=[
                pltpu.VMEM((2, _D, _F), jnp.float32),
                pltpu.VMEM((2, _F, _D), jnp.float32),
                pltpu.VMEM((_D, _F), jnp.bfloat16),
                pltpu.VMEM((_F, _D), jnp.bfloat16),
                pltpu.SemaphoreType.DMA((2,)),
                pltpu.SemaphoreType.DMA((2,)),
            ],
        ),
        out_shape=jax.ShapeDtypeStruct((_NPAD, _D), jnp.float32),
        compiler_params=pltpu.CompilerParams(
            dimension_semantics=("arbitrary",)),
    )
    osrt = ffn_call(te, act, stt, rid, ne, nv, xs, gs, w1, w2)

    y = sc_gather(osrt, p3)
    return y.reshape(Bc, Tc, Dc)
